# baseline (speedup 1.0000x reference)
import jax
import jax.numpy as jnp
from jax import lax
from jax.experimental import pallas as pl
from jax.experimental.pallas import tpu as pltpu, tpu_sc as plsc

WIDTH = 1000
FEATURE_DIM = 100000
N = 1024
TC_ROWS = 768
SC_ROWS = N - TC_ROWS
COL_BLOCK = 2048

NW = 32
ROWS_PER_W = SC_ROWS // NW
SLICE = ROWS_PER_W * FEATURE_DIM
ZBUF = 50000
NCHUNK = SLICE // ZBUF

_GATHER_DN = jax.lax.GatherDimensionNumbers(
    offset_dims=(), collapsed_slice_dims=(0,), start_index_map=(0,))


def _gather16(v, g):
    return jax.lax.gather(
        v, g[:, None], _GATHER_DN, (1,),
        mode=jax.lax.GatherScatterMode.PROMISE_IN_BOUNDS)


def _tc_block(state_ref, out_ref):
    j = pl.program_id(0)
    idx = state_ref[:, 0] + WIDTH * state_ref[:, 1]
    cols = jax.lax.broadcasted_iota(jnp.int32, out_ref.shape, 1) + j * COL_BLOCK
    out_ref[...] = (cols == idx[:, None]).astype(jnp.float32)


def _sc_body(state_hbm, out_hbm, zeros_v, state_v, idx_v, ones_v, sem, zsem):
    c = lax.axis_index("c")
    s = lax.axis_index("s")
    wid = s * 2 + c
    base_row = wid * ROWS_PER_W
    base_flat = base_row * FEATURE_DIM

    zv = jnp.zeros((16,), jnp.float32)

    def zloop(i, carry):
        zeros_v[pl.ds(i * 16, 16)] = zv
        return carry

    lax.fori_loop(0, ZBUF // 16, zloop, 0, unroll=8)

    pltpu.sync_copy(state_hbm.at[pl.ds(2 * base_row, 2 * ROWS_PER_W)], state_v)

    lane = lax.broadcasted_iota(jnp.int32, (16,), 0)
    even = (2 * lane) % 16
    odd = (2 * lane + 1) % 16
    v = state_v[...]
    xs = _gather16(v, even)
    ys = _gather16(v, odd)
    rows = lane % 8
    idx_v[...] = base_flat + rows * FEATURE_DIM + xs + WIDTH * ys
    ones_v[...] = jnp.ones((16,), jnp.float32)

    def floop(i, carry):
        pltpu.make_async_copy(
            zeros_v, out_hbm.at[pl.ds(base_flat + i * ZBUF, ZBUF)], zsem
        ).start()
        return carry

    lax.fori_loop(0, NCHUNK, floop, 0)

    def wloop(i, carry):
        pltpu.make_async_copy(
            zeros_v, out_hbm.at[pl.ds(base_flat + i * ZBUF, ZBUF)], zsem
        ).wait()
        return carry

    lax.fori_loop(0, NCHUNK, wloop, 0)

    pltpu.async_copy(ones_v, out_hbm.at[idx_v], sem).wait()


def kernel(state):
    tc_out = pl.pallas_call(
        _tc_block,
        grid=(pl.cdiv(FEATURE_DIM, COL_BLOCK),),
        in_specs=[pl.BlockSpec((TC_ROWS, 2), lambda j: (0, 0))],
        out_specs=pl.BlockSpec((TC_ROWS, COL_BLOCK), lambda j: (0, j)),
        out_shape=jax.ShapeDtypeStruct((TC_ROWS, FEATURE_DIM), jnp.float32),
    )(state[:TC_ROWS])

    sc_out = pl.kernel(
        _sc_body,
        out_type=jax.ShapeDtypeStruct((SC_ROWS * FEATURE_DIM,), jnp.float32),
        mesh=plsc.VectorSubcoreMesh(core_axis_name="c", subcore_axis_name="s"),
        scratch_types=[
            pltpu.VMEM((ZBUF,), jnp.float32),
            pltpu.VMEM((2 * ROWS_PER_W,), jnp.int32),
            pltpu.VMEM((16,), jnp.int32),
            pltpu.VMEM((16,), jnp.float32),
            pltpu.SemaphoreType.DMA,
            pltpu.SemaphoreType.DMA,
        ],
    )(state[TC_ROWS:].reshape(-1))

    return jnp.concatenate(
        [tc_out, sc_out.reshape(SC_ROWS, FEATURE_DIM)], axis=0)
